# Initial kernel scaffold; baseline (speedup 1.0000x reference)
import jax
import jax.numpy as jnp
from jax.experimental import pallas as pl


def kernel(encoded_cfg_nodes, identifiers_encodings, ast_node_type_idx, ast_node_identifier_idx, map_value_idx, map_key_idx, type_table, W_ident, W_enc, b_enc, W_gate, b_gate):
    raise NotImplementedError("write your pallas kernel here")


# speedup vs baseline: 1.7577x; 1.7577x over previous
import functools

import jax
import jax.numpy as jnp
from jax.experimental import pallas as pl

N_CFG = 60000
D = 256
BR = 1200


def _z_body(x_ref, w_ref, b_ref, o_ref):
    o_ref[...] = jax.nn.relu(
        jnp.dot(x_ref[...], w_ref[...], preferred_element_type=jnp.float32)
        + b_ref[...]
    )


def _gate_body(e_ref, n_ref, wg_ref, bg_ref, o_ref):
    e = e_ref[...]
    n = n_ref[...]
    ge = (
        jnp.dot(e, wg_ref[0:D, :], preferred_element_type=jnp.float32)
        + jnp.dot(n, wg_ref[D:2 * D, :], preferred_element_type=jnp.float32)
        + bg_ref[...]
    )
    g = jax.nn.sigmoid(ge)
    o_ref[...] = g * e + (1.0 - g) * n


@functools.partial(jax.jit, static_argnames=())
def _run(encoded_cfg_nodes, map_value_idx, map_key_idx, W_enc, b_enc, W_gate, b_gate):
    n_ast = 200000
    mv = map_value_idx.astype(jnp.int32)
    mk = map_key_idx.astype(jnp.int32)

    keyslot = jnp.zeros((n_ast,), jnp.int32).at[mv].set(mk)
    k = keyslot[mv]
    idx = jnp.full((N_CFG,), -1, jnp.int32).at[mk].set(k)
    mask = idx >= 0

    grid = (N_CFG // BR,)
    z = pl.pallas_call(
        _z_body,
        grid=grid,
        in_specs=[
            pl.BlockSpec((BR, D), lambda i: (i, 0)),
            pl.BlockSpec((D, D), lambda i: (0, 0)),
            pl.BlockSpec((1, D), lambda i: (0, 0)),
        ],
        out_specs=pl.BlockSpec((BR, D), lambda i: (i, 0)),
        out_shape=jax.ShapeDtypeStruct((N_CFG, D), jnp.float32),
    )(encoded_cfg_nodes, W_enc, b_enc.reshape(1, D))

    n = jnp.where(mask[:, None], z[jnp.maximum(idx, 0)], encoded_cfg_nodes)

    out = pl.pallas_call(
        _gate_body,
        grid=grid,
        in_specs=[
            pl.BlockSpec((BR, D), lambda i: (i, 0)),
            pl.BlockSpec((BR, D), lambda i: (i, 0)),
            pl.BlockSpec((2 * D, D), lambda i: (0, 0)),
            pl.BlockSpec((1, D), lambda i: (0, 0)),
        ],
        out_specs=pl.BlockSpec((BR, D), lambda i: (i, 0)),
        out_shape=jax.ShapeDtypeStruct((N_CFG, D), jnp.float32),
    )(encoded_cfg_nodes, n, W_gate, b_gate.reshape(1, D))
    return out


def kernel(encoded_cfg_nodes, identifiers_encodings, ast_node_type_idx,
           ast_node_identifier_idx, map_value_idx, map_key_idx,
           type_table, W_ident, W_enc, b_enc, W_gate, b_gate):
    return _run(encoded_cfg_nodes, map_value_idx, map_key_idx,
                W_enc, b_enc, W_gate, b_gate)


# speedup vs baseline: 3.7359x; 2.1255x over previous
import functools

import jax
import jax.numpy as jnp
from jax.experimental import pallas as pl

N_CFG = 60000
D = 256
BR = 1200


def _z_body(x_ref, w_ref, b_ref, o_ref):
    o_ref[...] = jax.nn.relu(
        jnp.dot(x_ref[...], w_ref[...], preferred_element_type=jnp.float32)
        + b_ref[...]
    )


def _gate_body(e_ref, n_ref, wg_ref, bg_ref, o_ref):
    e = e_ref[...]
    n = n_ref[...]
    ge = (
        jnp.dot(e, wg_ref[0:D, :], preferred_element_type=jnp.float32)
        + jnp.dot(n, wg_ref[D:2 * D, :], preferred_element_type=jnp.float32)
        + bg_ref[...]
    )
    g = jax.nn.sigmoid(ge)
    o_ref[...] = g * e + (1.0 - g) * n


@functools.partial(jax.jit, static_argnames=())
def _run(encoded_cfg_nodes, map_value_idx, map_key_idx, W_enc, b_enc, W_gate, b_gate):
    n_ast = 200000
    mv = map_value_idx.astype(jnp.int32)
    mk = map_key_idx.astype(jnp.int32)

    keyslot = jnp.zeros((n_ast,), jnp.int32).at[mv].set(mk)
    k = keyslot[mv]
    idx = jnp.full((N_CFG,), -1, jnp.int32).at[mk].set(k)
    mask = idx >= 0
    idx2 = jnp.where(mask, idx, jnp.arange(N_CFG, dtype=jnp.int32))

    grid = (N_CFG // BR,)
    z = pl.pallas_call(
        _z_body,
        grid=grid,
        in_specs=[
            pl.BlockSpec((BR, D), lambda i: (i, 0)),
            pl.BlockSpec((D, D), lambda i: (0, 0)),
            pl.BlockSpec((1, D), lambda i: (0, 0)),
        ],
        out_specs=pl.BlockSpec((BR, D), lambda i: (i, 0)),
        out_shape=jax.ShapeDtypeStruct((N_CFG, D), jnp.float32),
    )(encoded_cfg_nodes, W_enc, b_enc.reshape(1, D))

    n = jnp.where(mask[:, None], z[idx2], encoded_cfg_nodes)

    out = pl.pallas_call(
        _gate_body,
        grid=grid,
        in_specs=[
            pl.BlockSpec((BR, D), lambda i: (i, 0)),
            pl.BlockSpec((BR, D), lambda i: (i, 0)),
            pl.BlockSpec((2 * D, D), lambda i: (0, 0)),
            pl.BlockSpec((1, D), lambda i: (0, 0)),
        ],
        out_specs=pl.BlockSpec((BR, D), lambda i: (i, 0)),
        out_shape=jax.ShapeDtypeStruct((N_CFG, D), jnp.float32),
    )(encoded_cfg_nodes, n, W_gate, b_gate.reshape(1, D))
    return out


def kernel(encoded_cfg_nodes, identifiers_encodings, ast_node_type_idx,
           ast_node_identifier_idx, map_value_idx, map_key_idx,
           type_table, W_ident, W_enc, b_enc, W_gate, b_gate):
    return _run(encoded_cfg_nodes, map_value_idx, map_key_idx,
                W_enc, b_enc, W_gate, b_gate)


# speedup vs baseline: 11.2374x; 3.0079x over previous
import jax
import jax.numpy as jnp
from jax import lax
from jax.experimental import pallas as pl
from jax.experimental.pallas import tpu as pltpu
from jax.experimental.pallas import tpu_sc as plsc

N_AST = 200000
N_CFG = 60000
D = 256

NS = 16
NC = 2
NPAD = 65536
NEXTRA = NPAD - N_CFG
CHUNK = NPAD // NS
ROWS = CHUNK // 128

OCCT = 205568
KEYTS = 207616
KTRASH = 205568
RSPAN = 73728
OCC2T = 71168
IDXSP = 73216
MKPAD = NPAD
ITRASH = 71168

GROWS = NPAD // (NC * NS)
GBLK = 128
GNB = GROWS // GBLK

BR = 1200


def _win(ref, j, s):
    return ref.at[j, pl.ds(s * 16, 16)]


def _route_body(mv_hbm, mk_hbm, gw_hbm, kq_hbm,
                mv2, mk2, k2, lw2, vb2, ib2, post, tmp, zb, sbuf,
                occ, keyt, occ2):
    core = lax.axis_index("c")

    @pl.when(core == 0)
    def _route_core0():
        _route_inner(mv_hbm, mk_hbm, gw_hbm, kq_hbm,
                     mv2, mk2, k2, lw2, vb2, ib2, post, tmp, zb, sbuf,
                     occ, keyt, occ2)


def _route_inner(mv_hbm, mk_hbm, gw_hbm, kq_hbm,
                 mv2, mk2, k2, lw2, vb2, ib2, post, tmp, zb, sbuf,
                 occ, keyt, occ2):
    w = lax.axis_index("s")
    wb = w * CHUNK
    lane = lax.iota(jnp.int32, 16)
    wbit = lax.shift_left(jnp.int32(1), w)
    wthr = lax.shift_left(jnp.int32(2), w)

    def _fill(i, _):
        zb[pl.ds(i * 16, 16)] = jnp.zeros((16,), jnp.int32)
        return _
    lax.fori_loop(0, 128, _fill, 0)
    sbuf[pl.ds(16, 16)] = jnp.full((16,), -1, jnp.int32)

    for c in range(6):
        pltpu.sync_copy(zb.at[pl.ds(0, 2048)],
                        occ.at[pl.ds(w * 12848 + c * 2048, 2048)])
    pltpu.sync_copy(zb.at[pl.ds(0, 560)],
                    occ.at[pl.ds(w * 12848 + 12288, 560)])
    for c in range(2):
        pltpu.sync_copy(zb.at[pl.ds(0, 2048)],
                        occ2.at[pl.ds(w * 4448 + c * 2048, 2048)])
    pltpu.sync_copy(zb.at[pl.ds(0, 352)],
                    occ2.at[pl.ds(w * 4448 + 4096, 352)])

    pltpu.sync_copy(mv_hbm.at[pl.ds(w * ROWS, ROWS)], mv2)
    pltpu.sync_copy(mk_hbm.at[pl.ds(w * ROWS, ROWS)], mk2)
    plsc.subcore_barrier()

    for p in range(3):
        def _ascan(j, _):
            for s in range(8):
                t = _win(mv2, j, s)[...]
                winbase = wb + j * 128 + s * 16
                tr = t - p * RSPAN
                m = (tr >= 0) & (tr < RSPAN)
                tp = jnp.where(m, tr, RSPAN + lane)
                ks, ls = plsc.sort_key_val(tp * 16 + lane, lane)
                ts = ks >> 4
                sbuf[pl.ds(0, 16)] = ts
                nxt = sbuf[pl.ds(1, 16)]
                winm = (ts != nxt) & (ts < RSPAN)
                tq2 = jnp.where(winm, ts, 0)
                plsc.store_scatter(post, [tq2], winbase + ls, mask=winm)
            return _
        lax.fori_loop(0, ROWS, _ascan, 0)

        def _arescan(j, _):
            for s in range(8):
                t = _win(mv2, j, s)[...]
                gi = wb + j * 128 + s * 16 + lane
                tr = t - p * RSPAN
                m = (tr >= 0) & (tr < RSPAN)
                tq = jnp.where(m, tr, 0)
                g = plsc.load_gather(post, [tq], mask=m)
                lwc = m & (g == gi)
                if p == 0:
                    prev = jnp.zeros((16,), jnp.int32)
                else:
                    prev = _win(lw2, j, s)[...]
                _win(lw2, j, s)[...] = jnp.where(lwc, 1, prev)
            return _
        lax.fori_loop(0, ROWS, _arescan, 0)

    def _aocc(j, _):
        for s in range(8):
            _win(vb2, j, s)[...] = _win(lw2, j, s)[...] * wbit
        pltpu.sync_copy(vb2.at[j], occ.at[mv2.at[j]], add=True)
        return _
    lax.fori_loop(0, ROWS, _aocc, 0)
    plsc.subcore_barrier()

    def _akey(j, _):
        pltpu.sync_copy(occ.at[mv2.at[j]], tmp)
        for s in range(8):
            o = tmp[pl.ds(s * 16, 16)]
            gi = wb + j * 128 + s * 16 + lane
            gwm = (_win(lw2, j, s)[...] != 0) & (o < wthr)
            _win(ib2, j, s)[...] = jnp.where(
                gwm, _win(mv2, j, s)[...], KTRASH + (gi & 2047))
        pltpu.sync_copy(mk2.at[j], keyt.at[ib2.at[j]])
        return _
    lax.fori_loop(0, ROWS, _akey, 0)
    plsc.subcore_barrier()

    def _bgat(j, _):
        pltpu.sync_copy(keyt.at[mv2.at[j]], k2.at[j])
        return _
    lax.fori_loop(0, ROWS, _bgat, 0)

    def _cscan(j, _):
        for s in range(8):
            t = _win(mk2, j, s)[...]
            winbase = wb + j * 128 + s * 16
            ks, ls = plsc.sort_key_val(t * 16 + lane, lane)
            ts = ks >> 4
            sbuf[pl.ds(0, 16)] = ts
            nxt = sbuf[pl.ds(1, 16)]
            winm = ts != nxt
            plsc.store_scatter(post, [ts], winbase + ls, mask=winm)
        return _
    lax.fori_loop(0, ROWS, _cscan, 0)

    def _crescan(j, _):
        for s in range(8):
            t = _win(mk2, j, s)[...]
            gi = wb + j * 128 + s * 16 + lane
            g = plsc.load_gather(post, [t])
            _win(lw2, j, s)[...] = jnp.where(g == gi, 1, 0)
        return _
    lax.fori_loop(0, ROWS, _crescan, 0)

    def _cocc(j, _):
        for s in range(8):
            _win(vb2, j, s)[...] = _win(lw2, j, s)[...] * wbit
        pltpu.sync_copy(vb2.at[j], occ2.at[mk2.at[j]], add=True)
        return _
    lax.fori_loop(0, ROWS, _cocc, 0)
    plsc.subcore_barrier()

    def _cfin(j, _):
        pltpu.sync_copy(occ2.at[mk2.at[j]], tmp)
        for s in range(8):
            o = tmp[pl.ds(s * 16, 16)]
            gwm = (_win(lw2, j, s)[...] != 0) & (o < wthr)
            _win(lw2, j, s)[...] = jnp.where(gwm, 1, 0)
        return _
    lax.fori_loop(0, ROWS, _cfin, 0)
    pltpu.sync_copy(lw2, gw_hbm.at[pl.ds(w * ROWS, ROWS)])
    pltpu.sync_copy(k2, kq_hbm.at[pl.ds(w * ROWS, ROWS)])


_route = pl.kernel(
    _route_body,
    out_type=(
        jax.ShapeDtypeStruct((NPAD // 128, 128), jnp.int32),
        jax.ShapeDtypeStruct((NPAD // 128, 128), jnp.int32),
    ),
    mesh=plsc.VectorSubcoreMesh(core_axis_name="c", subcore_axis_name="s"),
    scratch_types=[
        pltpu.VMEM((ROWS, 128), jnp.int32),
        pltpu.VMEM((ROWS, 128), jnp.int32),
        pltpu.VMEM((ROWS, 128), jnp.int32),
        pltpu.VMEM((ROWS, 128), jnp.int32),
        pltpu.VMEM((ROWS, 128), jnp.int32),
        pltpu.VMEM((ROWS, 128), jnp.int32),
        pltpu.VMEM((RSPAN,), jnp.int32),
        pltpu.VMEM((128,), jnp.int32),
        pltpu.VMEM((2048,), jnp.int32),
        pltpu.VMEM((32,), jnp.int32),
        pltpu.VMEM_SHARED((OCCT,), jnp.int32),
        pltpu.VMEM_SHARED((KEYTS,), jnp.int32),
        pltpu.VMEM_SHARED((OCC2T,), jnp.int32),
    ],
    compiler_params=pltpu.CompilerParams(needs_layout_passes=False),
)


def _gather_body(z_hbm, mk_hbm, gw_hbm, kq_hbm, g_hbm, mcol_hbm,
                 mkc, gwc, kc, ibc, fb, zbf, onesf, idxb, mcolv,
                 buf0, buf1, sin0, sin1, sout0, sout1, idx2sp, mcolsp):
    c = lax.axis_index("c")
    s = lax.axis_index("s")
    wid = s * NC + c
    sb = s * CHUNK
    lane = lax.iota(jnp.int32, 16)

    def _fillf(i, _):
        zbf[pl.ds(i * 16, 16)] = jnp.zeros((16,), jnp.float32)
        return _
    lax.fori_loop(0, 128, _fillf, 0)
    for q in range(8):
        onesf[pl.ds(q * 16, 16)] = jnp.ones((16,), jnp.float32)

    def _defs(j, _):
        for q in range(8):
            v = sb + j * 128 + q * 16 + lane
            v = jnp.where(v < N_CFG, v, v - N_CFG)
            fb[pl.ds(j * 128 + q * 16, 16)] = v
        return _
    lax.fori_loop(0, ROWS, _defs, 0)
    pltpu.sync_copy(fb, idx2sp.at[pl.ds(sb, CHUNK)])
    pltpu.sync_copy(zbf.at[pl.ds(0, 2048)], mcolsp.at[pl.ds(sb, 2048)])
    pltpu.sync_copy(zbf.at[pl.ds(0, 2048)], mcolsp.at[pl.ds(sb + 2048, 2048)])

    pltpu.sync_copy(mk_hbm.at[pl.ds(s * ROWS, ROWS)], mkc)
    pltpu.sync_copy(gw_hbm.at[pl.ds(s * ROWS, ROWS)], gwc)
    pltpu.sync_copy(kq_hbm.at[pl.ds(s * ROWS, ROWS)], kc)
    plsc.subcore_barrier()

    def _scat(j, _):
        for q in range(8):
            gi = sb + j * 128 + q * 16 + lane
            gwm = _win(gwc, j, q)[...] != 0
            _win(ibc, j, q)[...] = jnp.where(
                gwm, _win(mkc, j, q)[...], ITRASH + (gi & 2047))
        pltpu.sync_copy(kc.at[j], idx2sp.at[ibc.at[j]])
        pltpu.sync_copy(onesf, mcolsp.at[ibc.at[j]])
        return _
    lax.fori_loop(0, ROWS, _scat, 0)
    plsc.subcore_barrier()

    pltpu.sync_copy(idx2sp.at[pl.ds(wid * GROWS, GROWS)], idxb)
    pltpu.sync_copy(mcolsp.at[pl.ds(wid * GROWS, GROWS)], mcolv)
    pltpu.sync_copy(mcolv, mcol_hbm.at[pl.ds(wid * GROWS, GROWS)])
    bufs = (buf0, buf1)
    sins = (sin0, sin1)
    souts = (sout0, sout1)
    outs = [None, None]
    for b in range(GNB):
        bi = b & 1
        if outs[bi] is not None:
            outs[bi].wait()
        pltpu.async_copy(
            z_hbm.at[idxb.at[pl.ds(b * GBLK, GBLK)]], bufs[bi], sins[bi]
        ).wait()
        outs[bi] = pltpu.async_copy(
            bufs[bi], g_hbm.at[pl.ds(wid * GROWS + b * GBLK, GBLK)], souts[bi])
    outs[0].wait()
    outs[1].wait()


_gather = pl.kernel(
    _gather_body,
    out_type=(
        jax.ShapeDtypeStruct((NPAD, D), jnp.float32),
        jax.ShapeDtypeStruct((NPAD,), jnp.float32),
    ),
    mesh=plsc.VectorSubcoreMesh(core_axis_name="c", subcore_axis_name="s"),
    scratch_types=[
        pltpu.VMEM((ROWS, 128), jnp.int32),
        pltpu.VMEM((ROWS, 128), jnp.int32),
        pltpu.VMEM((ROWS, 128), jnp.int32),
        pltpu.VMEM((ROWS, 128), jnp.int32),
        pltpu.VMEM((CHUNK,), jnp.int32),
        pltpu.VMEM((2048,), jnp.float32),
        pltpu.VMEM((128,), jnp.float32),
        pltpu.VMEM((GROWS,), jnp.int32),
        pltpu.VMEM((GROWS,), jnp.float32),
        pltpu.VMEM((GBLK, D), jnp.float32),
        pltpu.VMEM((GBLK, D), jnp.float32),
        pltpu.SemaphoreType.DMA,
        pltpu.SemaphoreType.DMA,
        pltpu.SemaphoreType.DMA,
        pltpu.SemaphoreType.DMA,
        pltpu.VMEM_SHARED((IDXSP,), jnp.int32),
        pltpu.VMEM_SHARED((IDXSP,), jnp.float32),
    ],
    compiler_params=pltpu.CompilerParams(needs_layout_passes=False),
)


def _z_body(x_ref, w_ref, b_ref, o_ref):
    o_ref[...] = jax.nn.relu(
        jnp.dot(x_ref[...], w_ref[...], preferred_element_type=jnp.float32)
        + b_ref[...]
    )


def _gate_body(e_ref, g_ref, m_ref, wg_ref, bg_ref, o_ref):
    e = e_ref[...]
    m = m_ref[...]
    n = m * g_ref[...] + (1.0 - m) * e
    ge = (
        jnp.dot(e, wg_ref[0:D, :], preferred_element_type=jnp.float32)
        + jnp.dot(n, wg_ref[D:2 * D, :], preferred_element_type=jnp.float32)
        + bg_ref[...]
    )
    g = jax.nn.sigmoid(ge)
    o_ref[...] = g * e + (1.0 - g) * n


@jax.jit
def _run(encoded_cfg_nodes, map_value_idx, map_key_idx, W_enc, b_enc, W_gate, b_gate):
    mv = map_value_idx.astype(jnp.int32)
    mk = map_key_idx.astype(jnp.int32)
    pad = jnp.arange(NEXTRA, dtype=jnp.int32)
    mv_pad = jnp.concatenate([mv, N_AST + pad]).reshape(NPAD // 128, 128)
    mk_pad = jnp.concatenate([mk, MKPAD + pad]).reshape(NPAD // 128, 128)

    gw, kq = _route(mv_pad, mk_pad)

    grid = (N_CFG // BR,)
    z = pl.pallas_call(
        _z_body,
        grid=grid,
        in_specs=[
            pl.BlockSpec((BR, D), lambda i: (i, 0)),
            pl.BlockSpec((D, D), lambda i: (0, 0)),
            pl.BlockSpec((1, D), lambda i: (0, 0)),
        ],
        out_specs=pl.BlockSpec((BR, D), lambda i: (i, 0)),
        out_shape=jax.ShapeDtypeStruct((N_CFG, D), jnp.float32),
    )(encoded_cfg_nodes, W_enc, b_enc.reshape(1, D))

    g_rows, mcol = _gather(z, mk_pad, gw, kq)

    out = pl.pallas_call(
        _gate_body,
        grid=grid,
        in_specs=[
            pl.BlockSpec((BR, D), lambda i: (i, 0)),
            pl.BlockSpec((BR, D), lambda i: (i, 0)),
            pl.BlockSpec((BR, 1), lambda i: (i, 0)),
            pl.BlockSpec((2 * D, D), lambda i: (0, 0)),
            pl.BlockSpec((1, D), lambda i: (0, 0)),
        ],
        out_specs=pl.BlockSpec((BR, D), lambda i: (i, 0)),
        out_shape=jax.ShapeDtypeStruct((N_CFG, D), jnp.float32),
    )(encoded_cfg_nodes, g_rows, mcol[:N_CFG].reshape(N_CFG, 1), W_gate,
      b_gate.reshape(1, D))
    return out


def kernel(encoded_cfg_nodes, identifiers_encodings, ast_node_type_idx,
           ast_node_identifier_idx, map_value_idx, map_key_idx,
           type_table, W_ident, W_enc, b_enc, W_gate, b_gate):
    return _run(encoded_cfg_nodes, map_value_idx, map_key_idx,
                W_enc, b_enc, W_gate, b_gate)
